# baseline (speedup 1.0000x reference)
import functools

import jax
import jax.numpy as jnp
import numpy as np
from jax import lax
from jax.experimental import pallas as pl
from jax.experimental.pallas import tpu as pltpu
from jax.experimental.pallas import tpu_sc as plsc

C = 768
H = 12
HD = 64
E = 16
NB, T = 2, 2048
N = NB * T
CAP = N // E
NSLOT = E * CAP
NSLOT_EXT = NSLOT + CAP
F = 4 * C
QT = 512
RT = 512


def _ln_qkv_body(x_ref, g_ref, b_ref, w_ref, o_ref):
    x = x_ref[...]
    m = jnp.mean(x, axis=1, keepdims=True)
    v = jnp.mean((x - m) * (x - m), axis=1, keepdims=True)
    xl = (x - m) / jnp.sqrt(v + 1e-5) * g_ref[...] + b_ref[...]
    o_ref[...] = lax.dot_general(xl, w_ref[...], (((1,), (1,)), ((), ())),
                                 preferred_element_type=jnp.float32)


def _ln_qkv(x2d, g, b, Wqkv):
    return pl.pallas_call(
        _ln_qkv_body,
        grid=(N // RT,),
        in_specs=[
            pl.BlockSpec((RT, C), lambda i: (i, 0)),
            pl.BlockSpec((1, C), lambda i: (0, 0)),
            pl.BlockSpec((1, C), lambda i: (0, 0)),
            pl.BlockSpec((3 * C, C), lambda i: (0, 0)),
        ],
        out_specs=pl.BlockSpec((RT, 3 * C), lambda i: (i, 0)),
        out_shape=jax.ShapeDtypeStruct((N, 3 * C), jnp.float32),
    )(x2d, g, b, Wqkv)


def _attn_body(q_ref, k_ref, v_ref, o_ref):
    q = q_ref[0]
    k = k_ref[0]
    v = v_ref[0]
    s = lax.dot_general(q, k, (((1,), (1,)), ((), ())),
                        preferred_element_type=jnp.float32)
    s = s * jnp.float32(1.0 / np.sqrt(HD))
    mx = jnp.max(s, axis=1, keepdims=True)
    p = jnp.exp(s - mx)
    p = p / jnp.sum(p, axis=1, keepdims=True)
    o_ref[0, 0] = lax.dot_general(p, v, (((1,), (0,)), ((), ())),
                                  preferred_element_type=jnp.float32)


def _attention(q, k, v):
    return pl.pallas_call(
        _attn_body,
        grid=(NB, H, T // QT),
        in_specs=[
            pl.BlockSpec((1, QT, HD), lambda b, h, qi: (h, b * (T // QT) + qi, 0)),
            pl.BlockSpec((1, T, HD), lambda b, h, qi: (h, b, 0)),
            pl.BlockSpec((1, T, HD), lambda b, h, qi: (h, b, 0)),
        ],
        out_specs=pl.BlockSpec((1, 1, QT, HD), lambda b, h, qi: (b, h, qi, 0)),
        out_shape=jax.ShapeDtypeStruct((NB, H, T, HD), jnp.float32),
    )(q, k, v)


def _router_body(x_ref, os_ref, wout_ref, g_ref, b_ref, wr_ref, br_ref,
                 wn_ref, bn_ref, nz_ref, x1_ref, xm_ref, eid_ref):
    att = lax.dot_general(os_ref[...], wout_ref[...], (((1,), (1,)), ((), ())),
                          preferred_element_type=jnp.float32)
    x1 = x_ref[...] + att
    x1_ref[...] = x1
    m = jnp.mean(x1, axis=1, keepdims=True)
    v = jnp.mean((x1 - m) * (x1 - m), axis=1, keepdims=True)
    xm = (x1 - m) / jnp.sqrt(v + 1e-5) * g_ref[...] + b_ref[...]
    xm_ref[...] = xm
    logits = lax.dot_general(xm, wr_ref[...], (((1,), (1,)), ((), ())),
                             preferred_element_type=jnp.float32) + br_ref[...]
    nl = lax.dot_general(xm, wn_ref[...], (((1,), (1,)), ((), ())),
                         preferred_element_type=jnp.float32) + bn_ref[...]
    noisy = logits + nz_ref[...] * jax.nn.softplus(nl)
    mx = jnp.max(noisy, axis=1, keepdims=True)
    iot = lax.broadcasted_iota(jnp.int32, noisy.shape, 1)
    eid = jnp.min(jnp.where(noisy == mx, iot, E), axis=1)
    eid_ref[0, 0] = eid


def _router(x2d, os2d, Wout, g, b, Wroute, broute, Wnoise, bnoise, noise2d):
    return pl.pallas_call(
        _router_body,
        grid=(N // RT,),
        in_specs=[
            pl.BlockSpec((RT, C), lambda i: (i, 0)),
            pl.BlockSpec((RT, C), lambda i: (i, 0)),
            pl.BlockSpec((C, C), lambda i: (0, 0)),
            pl.BlockSpec((1, C), lambda i: (0, 0)),
            pl.BlockSpec((1, C), lambda i: (0, 0)),
            pl.BlockSpec((E, C), lambda i: (0, 0)),
            pl.BlockSpec((1, E), lambda i: (0, 0)),
            pl.BlockSpec((E, C), lambda i: (0, 0)),
            pl.BlockSpec((1, E), lambda i: (0, 0)),
            pl.BlockSpec((RT, E), lambda i: (i, 0)),
        ],
        out_specs=[
            pl.BlockSpec((RT, C), lambda i: (i, 0)),
            pl.BlockSpec((RT, C), lambda i: (i, 0)),
            pl.BlockSpec((1, 1, RT), lambda i: (i, 0, 0)),
        ],
        out_shape=[
            jax.ShapeDtypeStruct((N, C), jnp.float32),
            jax.ShapeDtypeStruct((N, C), jnp.float32),
            jax.ShapeDtypeStruct((N // RT, 1, RT), jnp.int32),
        ],
    )(x2d, os2d, Wout, g, b, Wroute, broute, Wnoise, bnoise, noise2d)


def _dispatch_body(eid_hbm, gidx_hbm, slot_hbm, eid_v, gidx_v, slot_v, counts_v):
    cid = lax.axis_index("c")
    sid = lax.axis_index("s")

    @pl.when(jnp.logical_and(cid == 0, sid == 0))
    def _():
        pltpu.sync_copy(eid_hbm, eid_v)
        counts_v[...] = jnp.zeros((E,), jnp.int32)

        def zero_body(i, carry):
            gidx_v[pl.ds(i * 16, 16)] = jnp.zeros((16,), jnp.int32)
            return carry

        lax.fori_loop(0, NSLOT // 16, zero_body, 0)

        def body(i, carry):
            e16 = eid_v[pl.ds(i * 16, 16)]
            pc, _ = plsc.scan_count(e16)
            base = plsc.load_gather(counts_v, [e16])
            rank = base + pc - 1
            valid = rank < CAP
            tid = i * 16 + lax.iota(jnp.int32, 16)
            slot = jnp.where(valid, e16 * CAP + rank, NSLOT)
            plsc.store_scatter(gidx_v, [slot], tid, mask=valid)
            slot_v[pl.ds(i * 16, 16)] = slot
            plsc.addupdate_scatter(counts_v, [e16], jnp.ones((16,), jnp.int32))
            return carry

        lax.fori_loop(0, N // 16, body, 0)
        pltpu.sync_copy(gidx_v, gidx_hbm)
        pltpu.sync_copy(slot_v, slot_hbm)


def _dispatch(eid):
    mesh = plsc.VectorSubcoreMesh(core_axis_name="c", subcore_axis_name="s")
    return pl.kernel(
        _dispatch_body,
        out_type=(
            jax.ShapeDtypeStruct((NSLOT,), jnp.int32),
            jax.ShapeDtypeStruct((N,), jnp.int32),
        ),
        mesh=mesh,
        scratch_types=[
            pltpu.VMEM((N,), jnp.int32),
            pltpu.VMEM((NSLOT,), jnp.int32),
            pltpu.VMEM((N,), jnp.int32),
            pltpu.VMEM((E,), jnp.int32),
        ],
        compiler_params=pltpu.CompilerParams(needs_layout_passes=False),
    )(eid)


def _gather_body(rows, tab_hbm, idx_hbm, out_hbm, idx_v, rows_v, sem):
    cid = lax.axis_index("c")
    sid = lax.axis_index("s")
    wid = sid * 2 + cid
    base = wid * rows
    pltpu.sync_copy(idx_hbm.at[pl.ds(base, rows)], idx_v)
    pltpu.async_copy(tab_hbm.at[idx_v], rows_v, sem).wait()
    pltpu.sync_copy(rows_v, out_hbm.at[pl.ds(base, rows)])


def _gather_rows(table, idx, n_out):
    rows = n_out // 32
    mesh = plsc.VectorSubcoreMesh(core_axis_name="c", subcore_axis_name="s")
    return pl.kernel(
        functools.partial(_gather_body, rows),
        out_type=jax.ShapeDtypeStruct((n_out, C), jnp.float32),
        mesh=mesh,
        scratch_types=[
            pltpu.VMEM((rows,), jnp.int32),
            pltpu.VMEM((rows, C), jnp.float32),
            pltpu.SemaphoreType.DMA,
        ],
    )(table, idx)


def _ffn_body(xg_ref, w1_ref, b1_ref, w2_ref, b2_ref, y_ref):
    e = pl.program_id(0)
    xg = xg_ref[0].astype(jnp.bfloat16)
    h = lax.dot_general(xg, w1_ref[0], (((1,), (1,)), ((), ())),
                        preferred_element_type=jnp.float32) + b1_ref[0]
    h = jnp.maximum(h, 0.0).astype(jnp.bfloat16)
    y = lax.dot_general(h, w2_ref[0], (((1,), (1,)), ((), ())),
                        preferred_element_type=jnp.float32) + b2_ref[0]
    y_ref[0] = jnp.where(e == E, jnp.float32(0.0), y)


def _ffn(xg, W1, b1, W2, b2):
    xg3 = xg.reshape(E, CAP, C)
    y = pl.pallas_call(
        _ffn_body,
        grid=(E + 1,),
        in_specs=[
            pl.BlockSpec((1, CAP, C), lambda e: (jnp.minimum(e, E - 1), 0, 0)),
            pl.BlockSpec((1, F, C), lambda e: (jnp.minimum(e, E - 1), 0, 0)),
            pl.BlockSpec((1, 1, F), lambda e: (jnp.minimum(e, E - 1), 0, 0)),
            pl.BlockSpec((1, C, F), lambda e: (jnp.minimum(e, E - 1), 0, 0)),
            pl.BlockSpec((1, 1, C), lambda e: (jnp.minimum(e, E - 1), 0, 0)),
        ],
        out_specs=pl.BlockSpec((1, CAP, C), lambda e: (e, 0, 0)),
        out_shape=jax.ShapeDtypeStruct((E + 1, CAP, C), jnp.float32),
    )(xg3, W1.astype(jnp.bfloat16), b1.reshape(E, 1, F),
      W2.astype(jnp.bfloat16), b2.reshape(E, 1, C))
    return y.reshape(NSLOT_EXT, C)


def _add_body(a_ref, b_ref, o_ref):
    o_ref[...] = a_ref[...] + b_ref[...]


def _add(a, b):
    return pl.pallas_call(
        _add_body,
        grid=(N // RT,),
        in_specs=[
            pl.BlockSpec((RT, C), lambda i: (i, 0)),
            pl.BlockSpec((RT, C), lambda i: (i, 0)),
        ],
        out_specs=pl.BlockSpec((RT, C), lambda i: (i, 0)),
        out_shape=jax.ShapeDtypeStruct((N, C), jnp.float32),
    )(a, b)


def kernel(x, ln1_g, ln1_b, ln2_g, ln2_b, Wqkv, Wout, Wroute, broute, Wnoise,
           bnoise, W1, b1, W2, b2):
    x2d = x.reshape(N, C)
    qkv = _ln_qkv(x2d, ln1_g.reshape(1, C), ln1_b.reshape(1, C), Wqkv)
    q = qkv[:, :C].reshape(N, H, HD).transpose(1, 0, 2)
    k = qkv[:, C:2 * C].reshape(N, H, HD).transpose(1, 0, 2)
    v = qkv[:, 2 * C:].reshape(N, H, HD).transpose(1, 0, 2)
    o = _attention(q, k, v)
    o_s = jnp.transpose(o.reshape(NB, C, T), (0, 2, 1)).reshape(N, C)
    noise = jax.random.normal(jax.random.key(42), (NB, T, E),
                              jnp.float32).reshape(N, E)
    x1, xm, eid3 = _router(x2d, o_s, Wout, ln2_g.reshape(1, C),
                           ln2_b.reshape(1, C), Wroute, broute.reshape(1, E),
                           Wnoise, bnoise.reshape(1, E), noise)
    eid = eid3.reshape(N)
    gidx, slot = _dispatch(eid)
    xg = _gather_rows(xm, gidx, NSLOT)
    y = _ffn(xg, W1, b1, W2, b2)
    yg = _gather_rows(y, slot, N)
    out = _add(x1, yg)
    return out.reshape(NB, T, C)


# speedup vs baseline: 1.5669x; 1.5669x over previous
import functools

import jax
import jax.numpy as jnp
import numpy as np
from jax import lax
from jax.experimental import pallas as pl
from jax.experimental.pallas import tpu as pltpu
from jax.experimental.pallas import tpu_sc as plsc

C = 768
H = 12
HD = 64
E = 16
NB, T = 2, 2048
N = NB * T
CAP = N // E
NSLOT = E * CAP
NSLOT_EXT = NSLOT + CAP
F = 4 * C
QT = 512
RT = 512


def _ln_qkv_body(x_ref, g_ref, b_ref, w_ref, o_ref):
    x = x_ref[...]
    m = jnp.mean(x, axis=1, keepdims=True)
    v = jnp.mean((x - m) * (x - m), axis=1, keepdims=True)
    xl = (x - m) / jnp.sqrt(v + 1e-5) * g_ref[...] + b_ref[...]
    o_ref[...] = lax.dot_general(xl, w_ref[...], (((1,), (1,)), ((), ())),
                                 preferred_element_type=jnp.float32)


def _ln_qkv(x2d, g, b, Wqkv):
    return pl.pallas_call(
        _ln_qkv_body,
        grid=(N // RT,),
        in_specs=[
            pl.BlockSpec((RT, C), lambda i: (i, 0)),
            pl.BlockSpec((1, C), lambda i: (0, 0)),
            pl.BlockSpec((1, C), lambda i: (0, 0)),
            pl.BlockSpec((3 * C, C), lambda i: (0, 0)),
        ],
        out_specs=pl.BlockSpec((RT, 3 * C), lambda i: (i, 0)),
        out_shape=jax.ShapeDtypeStruct((N, 3 * C), jnp.float32),
    )(x2d, g, b, Wqkv)


def _attn_body(q_ref, k_ref, v_ref, o_ref):
    for u in range(2):
        q = q_ref[:, u * HD:(u + 1) * HD]
        k = k_ref[:, u * HD:(u + 1) * HD]
        v = v_ref[:, u * HD:(u + 1) * HD]
        s = lax.dot_general(q, k, (((1,), (1,)), ((), ())),
                            preferred_element_type=jnp.float32)
        s = s * jnp.float32(1.0 / np.sqrt(HD))
        mx = jnp.max(s, axis=1, keepdims=True)
        p = jnp.exp(s - mx)
        r = 1.0 / jnp.sum(p, axis=1, keepdims=True)
        o = lax.dot_general(p, v, (((1,), (0,)), ((), ())),
                            preferred_element_type=jnp.float32)
        o_ref[0, u] = o * r


def _attention(qkv2d):
    return pl.pallas_call(
        _attn_body,
        grid=(NB, H // 2, T // QT),
        in_specs=[
            pl.BlockSpec((QT, 2 * HD), lambda b, a, qi: (b * (T // QT) + qi, a)),
            pl.BlockSpec((T, 2 * HD), lambda b, a, qi: (b, H // 2 + a)),
            pl.BlockSpec((T, 2 * HD), lambda b, a, qi: (b, H + a)),
        ],
        out_specs=pl.BlockSpec((1, 2, QT, HD), lambda b, a, qi: (b, a, qi, 0)),
        out_shape=jax.ShapeDtypeStruct((NB, H, T, HD), jnp.float32),
    )(qkv2d, qkv2d, qkv2d)


def _router_body(x_ref, os_ref, wout_ref, g_ref, b_ref, wr_ref, br_ref,
                 wn_ref, bn_ref, nz_ref, x1_ref, xm_ref, eid_ref):
    o_s = jnp.transpose(os_ref[0], (1, 0))
    att = lax.dot_general(o_s, wout_ref[...], (((1,), (1,)), ((), ())),
                          preferred_element_type=jnp.float32)
    x1 = x_ref[...] + att
    x1_ref[...] = x1
    m = jnp.mean(x1, axis=1, keepdims=True)
    v = jnp.mean((x1 - m) * (x1 - m), axis=1, keepdims=True)
    xm = (x1 - m) / jnp.sqrt(v + 1e-5) * g_ref[...] + b_ref[...]
    xm_ref[...] = xm
    logits = lax.dot_general(xm, wr_ref[...], (((1,), (1,)), ((), ())),
                             preferred_element_type=jnp.float32) + br_ref[...]
    nl = lax.dot_general(xm, wn_ref[...], (((1,), (1,)), ((), ())),
                         preferred_element_type=jnp.float32) + bn_ref[...]
    noisy = logits + nz_ref[...] * jax.nn.softplus(nl)
    mx = jnp.max(noisy, axis=1, keepdims=True)
    iot = lax.broadcasted_iota(jnp.int32, noisy.shape, 1)
    eid = jnp.min(jnp.where(noisy == mx, iot, E), axis=1)
    eid_ref[0, 0] = eid


def _router(x2d, os2d, Wout, g, b, Wroute, broute, Wnoise, bnoise, noise2d):
    return pl.pallas_call(
        _router_body,
        grid=(N // RT,),
        in_specs=[
            pl.BlockSpec((RT, C), lambda i: (i, 0)),
            pl.BlockSpec((1, C, RT), lambda i: (i // (T // RT), 0, i % (T // RT))),
            pl.BlockSpec((C, C), lambda i: (0, 0)),
            pl.BlockSpec((1, C), lambda i: (0, 0)),
            pl.BlockSpec((1, C), lambda i: (0, 0)),
            pl.BlockSpec((E, C), lambda i: (0, 0)),
            pl.BlockSpec((1, E), lambda i: (0, 0)),
            pl.BlockSpec((E, C), lambda i: (0, 0)),
            pl.BlockSpec((1, E), lambda i: (0, 0)),
            pl.BlockSpec((RT, E), lambda i: (i, 0)),
        ],
        out_specs=[
            pl.BlockSpec((RT, C), lambda i: (i, 0)),
            pl.BlockSpec((RT, C), lambda i: (i, 0)),
            pl.BlockSpec((1, 1, RT), lambda i: (i, 0, 0)),
        ],
        out_shape=[
            jax.ShapeDtypeStruct((N, C), jnp.float32),
            jax.ShapeDtypeStruct((N, C), jnp.float32),
            jax.ShapeDtypeStruct((N // RT, 1, RT), jnp.int32),
        ],
    )(x2d, os2d, Wout, g, b, Wroute, broute, Wnoise, bnoise, noise2d)


def _dispatch_body(eid_hbm, gidx_hbm, slot_hbm, eid_v, gidx_v, slot_v, counts_v):
    cid = lax.axis_index("c")
    sid = lax.axis_index("s")

    @pl.when(jnp.logical_and(cid == 0, sid == 0))
    def _():
        pltpu.sync_copy(eid_hbm, eid_v)
        counts_v[...] = jnp.zeros((E,), jnp.int32)

        def zero_body(i, carry):
            gidx_v[pl.ds(i * 16, 16)] = jnp.zeros((16,), jnp.int32)
            return carry

        lax.fori_loop(0, NSLOT // 16, zero_body, 0)

        def body(i, carry):
            e16 = eid_v[pl.ds(i * 16, 16)]
            pc, _ = plsc.scan_count(e16)
            base = plsc.load_gather(counts_v, [e16])
            rank = base + pc - 1
            valid = rank < CAP
            tid = i * 16 + lax.iota(jnp.int32, 16)
            slot = jnp.where(valid, e16 * CAP + rank, NSLOT)
            plsc.store_scatter(gidx_v, [slot], tid, mask=valid)
            slot_v[pl.ds(i * 16, 16)] = slot
            plsc.addupdate_scatter(counts_v, [e16], jnp.ones((16,), jnp.int32))
            return carry

        lax.fori_loop(0, N // 16, body, 0)
        pltpu.sync_copy(gidx_v, gidx_hbm)
        pltpu.sync_copy(slot_v, slot_hbm)


def _dispatch(eid):
    mesh = plsc.VectorSubcoreMesh(core_axis_name="c", subcore_axis_name="s")
    return pl.kernel(
        _dispatch_body,
        out_type=(
            jax.ShapeDtypeStruct((NSLOT,), jnp.int32),
            jax.ShapeDtypeStruct((N,), jnp.int32),
        ),
        mesh=mesh,
        scratch_types=[
            pltpu.VMEM((N,), jnp.int32),
            pltpu.VMEM((NSLOT,), jnp.int32),
            pltpu.VMEM((N,), jnp.int32),
            pltpu.VMEM((E,), jnp.int32),
        ],
        compiler_params=pltpu.CompilerParams(needs_layout_passes=False),
    )(eid)


def _gather_body(rows, tab_hbm, idx_hbm, out_hbm, idx_v, rows_v, sem):
    cid = lax.axis_index("c")
    sid = lax.axis_index("s")
    wid = sid * 2 + cid
    base = wid * rows
    pltpu.sync_copy(idx_hbm.at[pl.ds(base, rows)], idx_v)
    pltpu.async_copy(tab_hbm.at[idx_v], rows_v, sem).wait()
    pltpu.sync_copy(rows_v, out_hbm.at[pl.ds(base, rows)])


def _gather_rows(table, idx, n_out):
    rows = n_out // 32
    mesh = plsc.VectorSubcoreMesh(core_axis_name="c", subcore_axis_name="s")
    return pl.kernel(
        functools.partial(_gather_body, rows),
        out_type=jax.ShapeDtypeStruct((n_out, C), jnp.float32),
        mesh=mesh,
        scratch_types=[
            pltpu.VMEM((rows,), jnp.int32),
            pltpu.VMEM((rows, C), jnp.float32),
            pltpu.SemaphoreType.DMA,
        ],
    )(table, idx)


def _ffn_body(xg_ref, w1_ref, b1_ref, w2_ref, b2_ref, y_ref):
    e = pl.program_id(0)
    xg = xg_ref[0]
    h = lax.dot_general(xg, w1_ref[0], (((1,), (1,)), ((), ())),
                        preferred_element_type=jnp.float32) + b1_ref[0]
    h = jnp.maximum(h, 0.0)
    y = lax.dot_general(h, w2_ref[0], (((1,), (1,)), ((), ())),
                        preferred_element_type=jnp.float32) + b2_ref[0]
    y_ref[0] = jnp.where(e == E, jnp.float32(0.0), y)


def _ffn(xg, W1, b1, W2, b2):
    xg3 = xg.reshape(E, CAP, C)
    y = pl.pallas_call(
        _ffn_body,
        grid=(E + 1,),
        in_specs=[
            pl.BlockSpec((1, CAP, C), lambda e: (jnp.minimum(e, E - 1), 0, 0)),
            pl.BlockSpec((1, F, C), lambda e: (jnp.minimum(e, E - 1), 0, 0)),
            pl.BlockSpec((1, 1, F), lambda e: (jnp.minimum(e, E - 1), 0, 0)),
            pl.BlockSpec((1, C, F), lambda e: (jnp.minimum(e, E - 1), 0, 0)),
            pl.BlockSpec((1, 1, C), lambda e: (jnp.minimum(e, E - 1), 0, 0)),
        ],
        out_specs=pl.BlockSpec((1, CAP, C), lambda e: (e, 0, 0)),
        out_shape=jax.ShapeDtypeStruct((E + 1, CAP, C), jnp.float32),
    )(xg3, W1, b1.reshape(E, 1, F), W2, b2.reshape(E, 1, C))
    return y.reshape(NSLOT_EXT, C)


def _add_body(a_ref, b_ref, o_ref):
    o_ref[...] = a_ref[...] + b_ref[...]


def _add(a, b):
    return pl.pallas_call(
        _add_body,
        grid=(N // RT,),
        in_specs=[
            pl.BlockSpec((RT, C), lambda i: (i, 0)),
            pl.BlockSpec((RT, C), lambda i: (i, 0)),
        ],
        out_specs=pl.BlockSpec((RT, C), lambda i: (i, 0)),
        out_shape=jax.ShapeDtypeStruct((N, C), jnp.float32),
    )(a, b)


def kernel(x, ln1_g, ln1_b, ln2_g, ln2_b, Wqkv, Wout, Wroute, broute, Wnoise,
           bnoise, W1, b1, W2, b2):
    x2d = x.reshape(N, C)
    qkv = _ln_qkv(x2d, ln1_g.reshape(1, C), ln1_b.reshape(1, C), Wqkv)
    o = _attention(qkv)
    o3 = o.reshape(NB, C, T)
    noise = jax.random.normal(jax.random.key(42), (NB, T, E),
                              jnp.float32).reshape(N, E)
    x1, xm, eid3 = _router(x2d, o3, Wout, ln2_g.reshape(1, C),
                           ln2_b.reshape(1, C), Wroute, broute.reshape(1, E),
                           Wnoise, bnoise.reshape(1, E), noise)
    eid = eid3.reshape(N)
    gidx, slot = _dispatch(eid)
    xg = _gather_rows(xm, gidx, NSLOT)
    y = _ffn(xg, W1, b1, W2, b2)
    yg = _gather_rows(y, slot, N)
    out = _add(x1, yg)
    return out.reshape(NB, T, C)
